# baseline (speedup 1.0000x reference)
import jax
import jax.numpy as jnp
from jax.experimental import pallas as pl
from jax.experimental.pallas import tpu as pltpu

K = 1024
D = 64
BLK = 1024
COMMITMENT_COST = 0.25


def _vq_block(x_ref, cb_ref, q_ref, loss_ref):
    i = pl.program_id(0)
    n_blocks = pl.num_programs(0)
    x = x_ref[...]
    cb = cb_ref[...]
    scores = jax.lax.dot_general(
        x, cb, (((1,), (1,)), ((), ())),
        preferred_element_type=jnp.float32,
    )
    x2 = jnp.sum(x * x, axis=1, keepdims=True)
    c2 = jnp.sum(cb * cb, axis=1, keepdims=True).T
    dist = x2 + c2 - 2.0 * scores
    min_d = jnp.min(dist, axis=1, keepdims=True)
    iota = jax.lax.broadcasted_iota(jnp.int32, dist.shape, 1)
    idx = jnp.min(jnp.where(dist == min_d, iota, K), axis=1, keepdims=True)
    onehot = (iota == idx).astype(jnp.float32)
    q = jax.lax.dot_general(
        onehot, cb, (((1,), (0,)), ((), ())),
        preferred_element_type=jnp.float32,
        precision=jax.lax.Precision.HIGHEST,
    )
    q_ref[...] = q
    diff = q - x
    part = jnp.sum(diff * diff).reshape(1, 1)

    @pl.when(i == 0)
    def _init():
        loss_ref[...] = jnp.zeros((1, 1), jnp.float32)

    loss_ref[...] += part

    @pl.when(i == n_blocks - 1)
    def _finalize():
        loss_ref[...] = loss_ref[...] * (
            COMMITMENT_COST / (n_blocks * BLK * D))


def kernel(x, codebook):
    n = x.shape[0] * x.shape[1]
    flat = x.reshape(n, D)
    q, loss = pl.pallas_call(
        _vq_block,
        grid=(n // BLK,),
        in_specs=[
            pl.BlockSpec((BLK, D), lambda i: (i, 0)),
            pl.BlockSpec((K, D), lambda i: (0, 0)),
        ],
        out_specs=[
            pl.BlockSpec((BLK, D), lambda i: (i, 0)),
            pl.BlockSpec((1, 1), lambda i: (0, 0)),
        ],
        out_shape=[
            jax.ShapeDtypeStruct((n, D), jnp.float32),
            jax.ShapeDtypeStruct((1, 1), jnp.float32),
        ],
    )(flat, codebook)
    return q.reshape(x.shape), loss[0, 0]


# speedup vs baseline: 1.7346x; 1.7346x over previous
import jax
import jax.numpy as jnp
from jax.experimental import pallas as pl
from jax.experimental.pallas import tpu as pltpu

K = 1024
D = 64
BLK = 1024
COMMITMENT_COST = 0.25


def _vq_block(x_ref, cb_ref, q_ref, loss_ref):
    i = pl.program_id(0)
    n_blocks = pl.num_programs(0)
    x = x_ref[...]
    cb = cb_ref[...]
    scores = jax.lax.dot_general(
        x, cb, (((1,), (1,)), ((), ())),
        preferred_element_type=jnp.float32,
    )
    x2 = jnp.sum(x * x, axis=1, keepdims=True)
    c2 = jnp.sum(cb * cb, axis=1, keepdims=True).T
    dist = x2 + c2 - 2.0 * scores

    val = dist
    idx = jax.lax.broadcasted_iota(jnp.int32, (BLK, K), 1)
    w = K
    while w > 128:
        h = w // 2
        a, b = val[:, :h], val[:, h:w]
        ia, ib = idx[:, :h], idx[:, h:w]
        take_a = a <= b
        val = jnp.where(take_a, a, b)
        idx = jnp.where(take_a, ia, ib)
        w = h
    min_d = jnp.min(val, axis=1, keepdims=True)
    big = jnp.int32(K)
    idx_col = jnp.min(jnp.where(val == min_d, idx, big), axis=1,
                      keepdims=True)

    iota = jax.lax.broadcasted_iota(jnp.int32, (BLK, K), 1)
    onehot = (iota == idx_col).astype(jnp.float32)
    q = jax.lax.dot_general(
        onehot, cb, (((1,), (0,)), ((), ())),
        preferred_element_type=jnp.float32,
    )
    q_ref[...] = q
    part = jnp.sum(min_d).reshape(1, 1)

    @pl.when(i == 0)
    def _init():
        loss_ref[...] = jnp.zeros((1, 1), jnp.float32)

    loss_ref[...] += part

    @pl.when(i == n_blocks - 1)
    def _finalize():
        loss_ref[...] = loss_ref[...] * (
            COMMITMENT_COST / (n_blocks * BLK * D))


def kernel(x, codebook):
    n = x.shape[0] * x.shape[1]
    flat = x.reshape(n, D)
    q, loss = pl.pallas_call(
        _vq_block,
        grid=(n // BLK,),
        in_specs=[
            pl.BlockSpec((BLK, D), lambda i: (i, 0)),
            pl.BlockSpec((K, D), lambda i: (0, 0)),
        ],
        out_specs=[
            pl.BlockSpec((BLK, D), lambda i: (i, 0)),
            pl.BlockSpec((1, 1), lambda i: (0, 0)),
        ],
        out_shape=[
            jax.ShapeDtypeStruct((n, D), jnp.float32),
            jax.ShapeDtypeStruct((1, 1), jnp.float32),
        ],
    )(flat, codebook)
    return q.reshape(x.shape), loss[0, 0]
